# baseline (speedup 1.0000x reference)
import functools

import jax
import jax.numpy as jnp
from jax import lax
from jax.experimental import pallas as pl
from jax.experimental.pallas import tpu as pltpu
from jax.experimental.pallas import tpu_sc as plsc

N = 10000
E = 160000
D = 256
NPAD = 10240
NC = 2
NS = 16
NW = NC * NS
K = 112
EPW = 5040
EPAD = NW * EPW
NBW = EPW // K
NSLOT = 3
DW = 16

_MESH = plsc.VectorSubcoreMesh(
    core_axis_name="c", subcore_axis_name="s", num_cores=NC, num_subcores=NS
)


HALF = 5120
TRASH = 5120
ACC_ROWS = 5128
RPT = HALF // NS
EPW2 = 2 * EPW
NBW2 = EPW2 // K


def _stage_edges(ei_hbm, sect, buf, w, pad_fn, nw, epw):
    base = sect + w * epw
    tail = E - (nw - 1) * epw
    npad = epw - tail

    @pl.when(w < nw - 1)
    def _():
        pltpu.sync_copy(ei_hbm.at[pl.ds(base, epw)], buf)

    @pl.when(w == nw - 1)
    def _():
        pltpu.sync_copy(ei_hbm.at[pl.ds(sect + (nw - 1) * epw, tail)],
                        buf.at[pl.ds(0, tail)])
        lanes = lax.iota(jnp.int32, 16)
        for j in range(npad // 16):
            buf[pl.ds(tail + j * 16, 16)] = pad_fn(lanes, j)


def _pad_src(lanes, j):
    return lanes * 16 + (j % 16)


def _pad_dst(lanes, j):
    return N + lanes * 15 + (j % 15)


@functools.partial(
    pl.kernel,
    out_type=jax.ShapeDtypeStruct((NPAD, DW), jnp.float32),
    mesh=_MESH,
    scratch_types=[
        pltpu.VMEM_SHARED((ACC_ROWS, DW), jnp.float32),
        pltpu.VMEM((EPW2,), jnp.int32),
        pltpu.VMEM((K, DW), jnp.float32),
        pltpu.VMEM((K,), jnp.int32),
        pltpu.VMEM((K,), jnp.int32),
        pltpu.VMEM((K,), jnp.int32),
        pltpu.SemaphoreType.DMA,
        pltpu.SemaphoreType.DMA,
        pltpu.SemaphoreType.DMA,
    ],
)
def _sc_degree(dst_hbm, ones_hbm, deg_hbm,
               acc, dst_v, ones_v, sidx0, sidx1, sidx2, sem0, sem1, sem2):
    sidx = (sidx0, sidx1, sidx2)
    sem = (sem0, sem1, sem2)
    c = lax.axis_index("c")
    t = lax.axis_index("s")
    lo = c * HALF
    pltpu.sync_copy(ones_hbm, acc.at[pl.ds(t * RPT, RPT)])
    _stage_edges(dst_hbm, E, dst_v, t, _pad_dst, NS, EPW2)
    pltpu.sync_copy(ones_hbm.at[pl.ds(0, K)], ones_v)
    plsc.subcore_barrier()

    def build(b, s):
        for v in range(K // 16):
            d16 = dst_v[pl.ds(b * K + v * 16, 16)]
            owned = (d16 >= lo) & (d16 < lo + HALF)
            sidx[s][pl.ds(v * 16, 16)] = jnp.where(owned, d16 - lo, TRASH)

    def sstart(s):
        pltpu.async_copy(ones_v, acc.at[sidx[s]], sem[s], add=True)

    def swait(s):
        pltpu.make_async_copy(ones_v, acc.at[sidx[s]], sem[s]).wait()

    for s in range(NSLOT):
        build(s, s)
        sstart(s)

    @pl.loop(0, NBW2 // NSLOT - 1)
    def _round(i):
        for s in range(NSLOT):
            swait(s)
            build(NSLOT * i + NSLOT + s, s)
            sstart(s)

    for s in range(NSLOT):
        swait(s)
    plsc.subcore_barrier()
    pltpu.sync_copy(
        acc.at[pl.ds(t * RPT, RPT)],
        deg_hbm.at[pl.ds(c * HALF + t * RPT, RPT)],
    )


@functools.partial(
    pl.kernel,
    out_type=jax.ShapeDtypeStruct((EPAD, D), jnp.float32),
    mesh=_MESH,
    scratch_types=[
        pltpu.VMEM((EPW,), jnp.int32),
        pltpu.VMEM((EPW,), jnp.int32),
        pltpu.VMEM((K,), jnp.int32),
        pltpu.VMEM((K,), jnp.int32),
        pltpu.VMEM((K,), jnp.int32),
        pltpu.VMEM((K,), jnp.int32),
        pltpu.VMEM((K,), jnp.int32),
        pltpu.VMEM((K,), jnp.int32),
        pltpu.VMEM((K, D), jnp.float32),
        pltpu.VMEM((K, D), jnp.float32),
        pltpu.VMEM((K, D), jnp.float32),
        pltpu.SemaphoreType.DMA,
        pltpu.SemaphoreType.DMA,
        pltpu.SemaphoreType.DMA,
        pltpu.SemaphoreType.DMA,
        pltpu.SemaphoreType.DMA,
        pltpu.SemaphoreType.DMA,
    ],
)
def _sc_gather(ei_hbm, hws_hbm, msgs_hbm,
               src_v, dst_v, gidx0, gidx1, gidx2, sidx0, sidx1, sidx2,
               rows0, rows1, rows2, gsem0, gsem1, gsem2,
               ssem0, ssem1, ssem2):
    gidx = (gidx0, gidx1, gidx2)
    sidx = (sidx0, sidx1, sidx2)
    rows = (rows0, rows1, rows2)
    gsem = (gsem0, gsem1, gsem2)
    ssem = (ssem0, ssem1, ssem2)
    w = lax.axis_index("c") * NS + lax.axis_index("s")
    base = w * EPW
    _stage_edges(ei_hbm, 0, src_v, w, _pad_src, NW, EPW)

    def gstart(b, s):
        for v in range(K // 16):
            gidx[s][pl.ds(v * 16, 16)] = src_v[pl.ds(b * K + v * 16, 16)]
        pltpu.async_copy(hws_hbm.at[gidx[s]], rows[s], gsem[s])

    def gwait(s):
        pltpu.make_async_copy(hws_hbm.at[gidx[s]], rows[s], gsem[s]).wait()

    def sstart(b, s):
        pltpu.async_copy(rows[s], msgs_hbm.at[pl.ds(base + b * K, K)],
                         ssem[s])

    def swait(b, s):
        pltpu.make_async_copy(rows[s], msgs_hbm.at[pl.ds(base + b * K, K)],
                              ssem[s]).wait()

    for s in range(NSLOT):
        gstart(s, s)

    @pl.loop(0, NBW // NSLOT - 1)
    def _round(i):
        for s in range(NSLOT):
            gwait(s)
            sstart(NSLOT * i + s, s)
        for s in range(NSLOT):
            swait(NSLOT * i + s, s)
            gstart(NSLOT * i + NSLOT + s, s)

    for s in range(NSLOT):
        gwait(s)
        sstart(NBW - NSLOT + s, s)
    for s in range(NSLOT):
        swait(NBW - NSLOT + s, s)


def _ln_relu(acc, hws, dinv_col, b, g, beta):
    t = (acc + hws) * dinv_col + b
    mu = jnp.mean(t, axis=-1, keepdims=True)
    var = jnp.mean((t - mu) ** 2, axis=-1, keepdims=True)
    tn = g * (t - mu) * lax.rsqrt(var + 1e-5) + beta
    return jnp.maximum(tn, 0.0)


def _enc_body(x_ref, wenc_ref, benc_ref, w1_ref, h0_ref, hw1_ref):
    h0 = jnp.maximum(
        jnp.dot(x_ref[...], wenc_ref[...], preferred_element_type=jnp.float32)
        + benc_ref[...], 0.0)
    h0_ref[...] = h0
    hw1_ref[...] = jnp.dot(
        h0, w1_ref[...], preferred_element_type=jnp.float32)


def _scale_body(deg_ref, hw1_ref, dinv_ref, hws1_ref):
    dinv = lax.rsqrt(deg_ref[...])
    dinv_ref[...] = dinv
    hws1_ref[...] = hw1_ref[...] * dinv[:, 0:1]


def _mid_body(acc_ref, hws_ref, hprev_ref, dinv_ref, b_ref, g_ref, beta_ref,
              w_ref, h_ref, hwsn_ref):
    dinv = dinv_ref[...][:, 0:1]
    h = _ln_relu(acc_ref[...], hws_ref[...], dinv,
                 b_ref[...], g_ref[...], beta_ref[...]) + hprev_ref[...]
    h_ref[...] = h
    hwsn_ref[...] = jnp.dot(
        h, w_ref[...], preferred_element_type=jnp.float32) * dinv


def _fin_body(acc_ref, hws_ref, hprev_ref, dinv_ref, b_ref, g_ref, beta_ref,
              w_ref, bout_ref, out_ref):
    dinv = dinv_ref[...][:, 0:1]
    h = _ln_relu(acc_ref[...], hws_ref[...], dinv,
                 b_ref[...], g_ref[...], beta_ref[...]) + hprev_ref[...]
    out_ref[...] = jnp.dot(
        h, w_ref[...], preferred_element_type=jnp.float32) + bout_ref[...]


_GRID = (NPAD // 320,)
_ROWS = pl.BlockSpec((320, D), lambda i: (i, 0))
_ROWS16 = pl.BlockSpec((320, DW), lambda i: (i, 0))
_WMAT = pl.BlockSpec((D, D), lambda i: (0, 0))
_VECB = pl.BlockSpec((1, D), lambda i: (0, 0))
_F32 = jnp.float32

_enc_call = pl.pallas_call(
    _enc_body,
    grid=_GRID,
    in_specs=[_ROWS, _WMAT, _VECB, _WMAT],
    out_specs=[_ROWS, _ROWS],
    out_shape=[
        jax.ShapeDtypeStruct((NPAD, D), _F32),
        jax.ShapeDtypeStruct((NPAD, D), _F32),
    ],
)

_scale_call = pl.pallas_call(
    _scale_body,
    grid=_GRID,
    in_specs=[_ROWS16, _ROWS],
    out_specs=[_ROWS16, _ROWS],
    out_shape=[
        jax.ShapeDtypeStruct((NPAD, DW), _F32),
        jax.ShapeDtypeStruct((NPAD, D), _F32),
    ],
)

_mid_call = pl.pallas_call(
    _mid_body,
    grid=_GRID,
    in_specs=[_ROWS, _ROWS, _ROWS, _ROWS16, _VECB, _VECB, _VECB, _WMAT],
    out_specs=[_ROWS, _ROWS],
    out_shape=[
        jax.ShapeDtypeStruct((NPAD, D), _F32),
        jax.ShapeDtypeStruct((NPAD, D), _F32),
    ],
)

_fin_call = pl.pallas_call(
    _fin_body,
    grid=_GRID,
    in_specs=[_ROWS, _ROWS, _ROWS, _ROWS16, _VECB, _VECB, _VECB, _WMAT, _VECB],
    out_specs=_ROWS,
    out_shape=jax.ShapeDtypeStruct((N, D), _F32),
)


def kernel(x, edge_index, W_enc, b_enc, W1, b1, g1, beta1,
           W2, b2, g2, beta2, W_out, b_out):
    ei_flat = edge_index.reshape(2 * E)
    dst = edge_index[1]
    deg1 = jnp.ones((N,), jnp.float32).at[dst].add(1.0)
    deg = jnp.broadcast_to(
        jnp.pad(deg1, (0, NPAD - N), constant_values=1.0)[:, None],
        (NPAD, DW))

    h0, hw1 = _enc_call(x, W_enc, b_enc.reshape(1, D), W1)
    dinv, hws1 = _scale_call(deg, hw1)
    msgs1 = _sc_gather(ei_flat, hws1)
    acc1 = jnp.zeros((NPAD, D), jnp.float32).at[dst].add(msgs1[:E])
    h1, hws2 = _mid_call(
        acc1, hws1, h0, dinv, b1.reshape(1, D), g1.reshape(1, D),
        beta1.reshape(1, D), W2)
    msgs2 = _sc_gather(ei_flat, hws2)
    acc2 = jnp.zeros((NPAD, D), jnp.float32).at[dst].add(msgs2[:E])
    return _fin_call(
        acc2, hws2, h1, dinv, b2.reshape(1, D), g2.reshape(1, D),
        beta2.reshape(1, D), W_out, b_out.reshape(1, D))


# speedup vs baseline: 1.0462x; 1.0462x over previous
import functools

import jax
import jax.numpy as jnp
from jax import lax
from jax.experimental import pallas as pl
from jax.experimental.pallas import tpu as pltpu
from jax.experimental.pallas import tpu_sc as plsc

N = 10000
E = 160000
D = 256
NPAD = 10240
NC = 2
NS = 16
NW = NC * NS
K = 112
EPW = 5040
EPAD = NW * EPW
NBW = EPW // K
NSLOT = 3
DW = 16

_MESH = plsc.VectorSubcoreMesh(
    core_axis_name="c", subcore_axis_name="s", num_cores=NC, num_subcores=NS
)


HALF = 5120
TRASH = 5120
ACC_ROWS = 5128
RPT = HALF // NS
EPW2 = 2 * EPW
NBW2 = EPW2 // K


def _stage_edges(ei_hbm, sect, buf, w, pad_fn, nw, epw):
    base = sect + w * epw
    tail = E - (nw - 1) * epw
    npad = epw - tail

    @pl.when(w < nw - 1)
    def _():
        pltpu.sync_copy(ei_hbm.at[pl.ds(base, epw)], buf)

    @pl.when(w == nw - 1)
    def _():
        pltpu.sync_copy(ei_hbm.at[pl.ds(sect + (nw - 1) * epw, tail)],
                        buf.at[pl.ds(0, tail)])
        lanes = lax.iota(jnp.int32, 16)
        for j in range(npad // 16):
            buf[pl.ds(tail + j * 16, 16)] = pad_fn(lanes, j)


def _pad_src(lanes, j):
    return lanes * 16 + (j % 16)


def _pad_dst(lanes, j):
    return N + lanes * 15 + (j % 15)


@functools.partial(
    pl.kernel,
    out_type=jax.ShapeDtypeStruct((NPAD, DW), jnp.float32),
    mesh=_MESH,
    scratch_types=[
        pltpu.VMEM_SHARED((ACC_ROWS, DW), jnp.float32),
        pltpu.VMEM((EPW2,), jnp.int32),
        pltpu.VMEM((K, DW), jnp.float32),
        pltpu.VMEM((K,), jnp.int32),
        pltpu.VMEM((K,), jnp.int32),
        pltpu.VMEM((K,), jnp.int32),
        pltpu.SemaphoreType.DMA,
        pltpu.SemaphoreType.DMA,
        pltpu.SemaphoreType.DMA,
    ],
)
def _sc_degree(dst_hbm, ones_hbm, deg_hbm,
               acc, dst_v, ones_v, sidx0, sidx1, sidx2, sem0, sem1, sem2):
    sidx = (sidx0, sidx1, sidx2)
    sem = (sem0, sem1, sem2)
    c = lax.axis_index("c")
    t = lax.axis_index("s")
    lo = c * HALF
    pltpu.sync_copy(ones_hbm, acc.at[pl.ds(t * RPT, RPT)])
    _stage_edges(dst_hbm, E, dst_v, t, _pad_dst, NS, EPW2)
    pltpu.sync_copy(ones_hbm.at[pl.ds(0, K)], ones_v)
    plsc.subcore_barrier()

    def build(b, s):
        for v in range(K // 16):
            d16 = dst_v[pl.ds(b * K + v * 16, 16)]
            owned = (d16 >= lo) & (d16 < lo + HALF)
            sidx[s][pl.ds(v * 16, 16)] = jnp.where(owned, d16 - lo, TRASH)

    def sstart(s):
        pltpu.async_copy(ones_v, acc.at[sidx[s]], sem[s], add=True)

    def swait(s):
        pltpu.make_async_copy(ones_v, acc.at[sidx[s]], sem[s]).wait()

    for s in range(NSLOT):
        build(s, s)
        sstart(s)

    @pl.loop(0, NBW2 // NSLOT - 1)
    def _round(i):
        for s in range(NSLOT):
            swait(s)
            build(NSLOT * i + NSLOT + s, s)
            sstart(s)

    for s in range(NSLOT):
        swait(s)
    plsc.subcore_barrier()
    pltpu.sync_copy(
        acc.at[pl.ds(t * RPT, RPT)],
        deg_hbm.at[pl.ds(c * HALF + t * RPT, RPT)],
    )


@functools.partial(
    pl.kernel,
    out_type=jax.ShapeDtypeStruct((EPAD, D), jnp.float32),
    mesh=_MESH,
    scratch_types=[
        pltpu.VMEM((EPW,), jnp.int32),
        pltpu.VMEM((EPW,), jnp.int32),
        pltpu.VMEM((K,), jnp.int32),
        pltpu.VMEM((K,), jnp.int32),
        pltpu.VMEM((K,), jnp.int32),
        pltpu.VMEM((K,), jnp.int32),
        pltpu.VMEM((K,), jnp.int32),
        pltpu.VMEM((K,), jnp.int32),
        pltpu.VMEM((K, D), jnp.float32),
        pltpu.VMEM((K, D), jnp.float32),
        pltpu.VMEM((K, D), jnp.float32),
        pltpu.SemaphoreType.DMA,
        pltpu.SemaphoreType.DMA,
        pltpu.SemaphoreType.DMA,
        pltpu.SemaphoreType.DMA,
        pltpu.SemaphoreType.DMA,
        pltpu.SemaphoreType.DMA,
    ],
)
def _sc_gather(ei_hbm, hws_hbm, msgs_hbm,
               src_v, dst_v, gidx0, gidx1, gidx2, sidx0, sidx1, sidx2,
               rows0, rows1, rows2, gsem0, gsem1, gsem2,
               ssem0, ssem1, ssem2):
    gidx = (gidx0, gidx1, gidx2)
    sidx = (sidx0, sidx1, sidx2)
    rows = (rows0, rows1, rows2)
    gsem = (gsem0, gsem1, gsem2)
    ssem = (ssem0, ssem1, ssem2)
    w = lax.axis_index("c") * NS + lax.axis_index("s")
    base = w * EPW
    _stage_edges(ei_hbm, 0, src_v, w, _pad_src, NW, EPW)

    def gstart(b, s):
        for v in range(K // 16):
            gidx[s][pl.ds(v * 16, 16)] = src_v[pl.ds(b * K + v * 16, 16)]
        pltpu.async_copy(hws_hbm.at[gidx[s]], rows[s], gsem[s])

    def gwait(s):
        pltpu.make_async_copy(hws_hbm.at[gidx[s]], rows[s], gsem[s]).wait()

    def sstart(b, s):
        pltpu.async_copy(rows[s], msgs_hbm.at[pl.ds(base + b * K, K)],
                         ssem[s])

    def swait(b, s):
        pltpu.make_async_copy(rows[s], msgs_hbm.at[pl.ds(base + b * K, K)],
                              ssem[s]).wait()

    for s in range(NSLOT):
        gstart(s, s)

    @pl.loop(0, NBW // NSLOT - 1)
    def _round(i):
        for s in range(NSLOT):
            gwait(s)
            sstart(NSLOT * i + s, s)
        for s in range(NSLOT):
            swait(NSLOT * i + s, s)
            gstart(NSLOT * i + NSLOT + s, s)

    for s in range(NSLOT):
        gwait(s)
        sstart(NBW - NSLOT + s, s)
    for s in range(NSLOT):
        swait(NBW - NSLOT + s, s)


def _ln_relu(acc, hws, dinv_col, b, g, beta):
    t = (acc + hws) * dinv_col + b
    mu = jnp.mean(t, axis=-1, keepdims=True)
    var = jnp.mean((t - mu) ** 2, axis=-1, keepdims=True)
    tn = g * (t - mu) * lax.rsqrt(var + 1e-5) + beta
    return jnp.maximum(tn, 0.0)


def _enc_body(x_ref, wenc_ref, benc_ref, w1_ref, h0_ref, hw1_ref):
    h0 = jnp.maximum(
        jnp.dot(x_ref[...], wenc_ref[...], preferred_element_type=jnp.float32)
        + benc_ref[...], 0.0)
    h0_ref[...] = h0
    hw1_ref[...] = jnp.dot(
        h0, w1_ref[...], preferred_element_type=jnp.float32)


def _scale_body(deg_ref, hw1_ref, dinv_ref, hws1_ref):
    dinv = lax.rsqrt(deg_ref[...])
    dinv_ref[...] = dinv
    hws1_ref[...] = hw1_ref[...] * dinv[:, 0:1]


def _mid_body(acc_ref, hws_ref, hprev_ref, dinv_ref, b_ref, g_ref, beta_ref,
              w_ref, h_ref, hwsn_ref):
    dinv = dinv_ref[...][:, 0:1]
    h = _ln_relu(acc_ref[...], hws_ref[...], dinv,
                 b_ref[...], g_ref[...], beta_ref[...]) + hprev_ref[...]
    h_ref[...] = h
    hwsn_ref[...] = jnp.dot(
        h, w_ref[...], preferred_element_type=jnp.float32) * dinv


def _fin_body(acc_ref, hws_ref, hprev_ref, dinv_ref, b_ref, g_ref, beta_ref,
              w_ref, bout_ref, out_ref):
    dinv = dinv_ref[...][:, 0:1]
    h = _ln_relu(acc_ref[...], hws_ref[...], dinv,
                 b_ref[...], g_ref[...], beta_ref[...]) + hprev_ref[...]
    out_ref[...] = jnp.dot(
        h, w_ref[...], preferred_element_type=jnp.float32) + bout_ref[...]


_GRID = (NPAD // 320,)
_ROWS = pl.BlockSpec((320, D), lambda i: (i, 0))
_ROWS16 = pl.BlockSpec((320, DW), lambda i: (i, 0))
_WMAT = pl.BlockSpec((D, D), lambda i: (0, 0))
_VECB = pl.BlockSpec((1, D), lambda i: (0, 0))
_F32 = jnp.float32

_enc_call = pl.pallas_call(
    _enc_body,
    grid=_GRID,
    in_specs=[_ROWS, _WMAT, _VECB, _WMAT],
    out_specs=[_ROWS, _ROWS],
    out_shape=[
        jax.ShapeDtypeStruct((NPAD, D), _F32),
        jax.ShapeDtypeStruct((NPAD, D), _F32),
    ],
)

_scale_call = pl.pallas_call(
    _scale_body,
    grid=_GRID,
    in_specs=[_ROWS16, _ROWS],
    out_specs=[_ROWS16, _ROWS],
    out_shape=[
        jax.ShapeDtypeStruct((NPAD, DW), _F32),
        jax.ShapeDtypeStruct((NPAD, D), _F32),
    ],
)

_mid_call = pl.pallas_call(
    _mid_body,
    grid=_GRID,
    in_specs=[_ROWS, _ROWS, _ROWS, _ROWS16, _VECB, _VECB, _VECB, _WMAT],
    out_specs=[_ROWS, _ROWS],
    out_shape=[
        jax.ShapeDtypeStruct((NPAD, D), _F32),
        jax.ShapeDtypeStruct((NPAD, D), _F32),
    ],
)

_fin_call = pl.pallas_call(
    _fin_body,
    grid=_GRID,
    in_specs=[_ROWS, _ROWS, _ROWS, _ROWS16, _VECB, _VECB, _VECB, _WMAT, _VECB],
    out_specs=_ROWS,
    out_shape=jax.ShapeDtypeStruct((N, D), _F32),
)


def kernel(x, edge_index, W_enc, b_enc, W1, b1, g1, beta1,
           W2, b2, g2, beta2, W_out, b_out):
    dst_s, src_s = jax.lax.sort([edge_index[1], edge_index[0]], num_keys=1)
    ei_flat = jnp.concatenate([src_s, dst_s])
    deg1 = jnp.ones((N,), jnp.float32).at[dst_s].add(
        1.0, indices_are_sorted=True)
    deg = jnp.broadcast_to(
        jnp.pad(deg1, (0, NPAD - N), constant_values=1.0)[:, None],
        (NPAD, DW))

    h0, hw1 = _enc_call(x, W_enc, b_enc.reshape(1, D), W1)
    dinv, hws1 = _scale_call(deg, hw1)
    msgs1 = _sc_gather(ei_flat, hws1)
    acc1 = jnp.zeros((NPAD, D), jnp.float32).at[dst_s].add(
        msgs1[:E], indices_are_sorted=True)
    h1, hws2 = _mid_call(
        acc1, hws1, h0, dinv, b1.reshape(1, D), g1.reshape(1, D),
        beta1.reshape(1, D), W2)
    msgs2 = _sc_gather(ei_flat, hws2)
    acc2 = jnp.zeros((NPAD, D), jnp.float32).at[dst_s].add(
        msgs2[:E], indices_are_sorted=True)
    return _fin_call(
        acc2, hws2, h1, dinv, b2.reshape(1, D), g2.reshape(1, D),
        beta2.reshape(1, D), W_out, b_out.reshape(1, D))
